# baseline (speedup 1.0000x reference)
import functools

import jax
import jax.numpy as jnp
from jax import lax
from jax.experimental import pallas as pl
from jax.experimental.pallas import tpu as pltpu
from jax.experimental.pallas import tpu_sc as plsc

L = 16
VREGS = 16
ROW = 224
NC = 2
NS = 16
NW = NC * NS
R = 8 * 96 * 224
ROWS_PER_W = R // NW
CHUNK = 64
NCHUNKS = ROWS_PER_W // CHUNK


_INF = object()


def _sort16(v, descending):
    if v is _INF:
        return _INF
    if descending:
        return plsc.sort_key_val(v, v, descending=True)[0]
    return jnp.sort(v)


def _smin(a, b):
    if a is _INF:
        return b
    if b is _INF:
        return a
    return jnp.minimum(a, b)


def _smax(a, b):
    if a is _INF or b is _INF:
        return _INF
    return jnp.maximum(a, b)


def _sort_row_vregs(vs):
    vs = list(vs)
    for v in range(VREGS):
        vs[v] = _sort16(vs[v], descending=(v & 1) == 1)
    for K in (2, 4, 8, 16):
        J = K // 2
        while J >= 1:
            for v in range(VREGS):
                if v & J == 0:
                    p = v | J
                    a, b = vs[v], vs[p]
                    lo = _smin(a, b)
                    hi = _smax(a, b)
                    if (v & K) == 0:
                        vs[v], vs[p] = lo, hi
                    else:
                        vs[v], vs[p] = hi, lo
            J //= 2
        for v in range(VREGS):
            vs[v] = _sort16(vs[v], descending=(v & K) != 0)
    assert all(vs[i] is not _INF for i in range(ROW // L))
    assert all(vs[i] is _INF for i in range(ROW // L, VREGS))
    return vs


def _sc_body(x_hbm, out_hbm, in0, in1, out0, out1, si0, si1, so0, so1):
    c = lax.axis_index("c")
    s = lax.axis_index("s")
    wid = s * NC + c
    base = wid * ROWS_PER_W
    ins, outs = (in0, in1), (out0, out1)
    sis, sos = (si0, si1), (so0, so1)

    def cs(idx):
        return pl.ds(base + idx * CHUNK, CHUNK)

    pltpu.async_copy(x_hbm.at[cs(0)], in0, si0)

    def outer_body(oi, carry):
        for b in range(2):
            idx = oi * 2 + b
            nb = 1 - b

            @pl.when(idx + 1 < NCHUNKS)
            def _():
                pltpu.async_copy(x_hbm.at[cs(idx + 1)], ins[nb], sis[nb])

            pltpu.make_async_copy(x_hbm.at[cs(idx)], ins[b], sis[b]).wait()

            @pl.when(idx >= 2)
            def _():
                pltpu.make_async_copy(outs[b], out_hbm.at[cs(idx - 2)],
                                      sos[b]).wait()

            @plsc.parallel_loop(0, CHUNK, 1, unroll=2)
            def row_body(r):
                vs = [ins[b][r, pl.ds(i * L, L)] for i in range(ROW // L)]
                vs += [_INF, _INF]
                vs = _sort_row_vregs(vs)
                for i in range(ROW // L):
                    outs[b][r, pl.ds(i * L, L)] = vs[i]
            pltpu.async_copy(outs[b], out_hbm.at[cs(idx)], sos[b])
        return carry

    lax.fori_loop(0, NCHUNKS // 2, outer_body, 0)
    pltpu.make_async_copy(out0, out_hbm.at[cs(NCHUNKS - 2)], so0).wait()
    pltpu.make_async_copy(out1, out_hbm.at[cs(NCHUNKS - 1)], so1).wait()


@jax.jit
def kernel(x):
    x2 = x.reshape(R, ROW)
    mesh = plsc.VectorSubcoreMesh(core_axis_name="c", subcore_axis_name="s")
    out = pl.kernel(
        _sc_body,
        out_type=jax.ShapeDtypeStruct((R, ROW), jnp.float32),
        mesh=mesh,
        scratch_types=[
            pltpu.VMEM((CHUNK, ROW), jnp.float32),
            pltpu.VMEM((CHUNK, ROW), jnp.float32),
            pltpu.VMEM((CHUNK, ROW), jnp.float32),
            pltpu.VMEM((CHUNK, ROW), jnp.float32),
            pltpu.SemaphoreType.DMA,
            pltpu.SemaphoreType.DMA,
            pltpu.SemaphoreType.DMA,
            pltpu.SemaphoreType.DMA,
        ],
        compiler_params=pltpu.CompilerParams(needs_layout_passes=False),
    )(x2)
    return out.reshape(x.shape)


# speedup vs baseline: 1.0729x; 1.0729x over previous
import functools

import jax
import jax.numpy as jnp
from jax import lax
from jax.experimental import pallas as pl
from jax.experimental.pallas import tpu as pltpu
from jax.experimental.pallas import tpu_sc as plsc

L = 16
VREGS = 16
ROW = 224
NC = 2
NS = 16
NW = NC * NS
R = 8 * 96 * 224
ROWS_PER_W = R // NW
CHUNK = 64
NCHUNKS = ROWS_PER_W // CHUNK


_INF = object()


def _sort16(v, descending):
    if v is _INF:
        return _INF
    if descending:
        return plsc.sort_key_val(v, v, descending=True)[0]
    return jnp.sort(v)


def _smin(a, b):
    if a is _INF:
        return b
    if b is _INF:
        return a
    return jnp.minimum(a, b)


def _smax(a, b):
    if a is _INF or b is _INF:
        return _INF
    return jnp.maximum(a, b)


def _sort_row_vregs(vs):
    vs = list(vs)
    for v in range(VREGS):
        vs[v] = _sort16(vs[v], descending=(v & 1) == 1)
    for K in (2, 4, 8, 16):
        J = K // 2
        while J >= 1:
            for v in range(VREGS):
                if v & J == 0:
                    p = v | J
                    a, b = vs[v], vs[p]
                    lo = _smin(a, b)
                    hi = _smax(a, b)
                    if (v & K) == 0:
                        vs[v], vs[p] = lo, hi
                    else:
                        vs[v], vs[p] = hi, lo
            J //= 2
        for v in range(VREGS):
            vs[v] = _sort16(vs[v], descending=(v & K) != 0)
    assert all(vs[i] is not _INF for i in range(ROW // L))
    assert all(vs[i] is _INF for i in range(ROW // L, VREGS))
    return vs


def _sc_body(x_hbm, out_hbm, in0, in1, out0, out1, si0, si1, so0, so1):
    c = lax.axis_index("c")
    s = lax.axis_index("s")
    wid = s * NC + c
    base = wid * ROWS_PER_W
    ins, outs = (in0, in1), (out0, out1)
    sis, sos = (si0, si1), (so0, so1)

    def cs(idx):
        return pl.ds(base + idx * CHUNK, CHUNK)

    pltpu.async_copy(x_hbm.at[cs(0)], in0, si0)

    def outer_body(oi, carry):
        for b in range(2):
            idx = oi * 2 + b
            nb = 1 - b

            @pl.when(idx + 1 < NCHUNKS)
            def _():
                pltpu.async_copy(x_hbm.at[cs(idx + 1)], ins[nb], sis[nb])

            pltpu.make_async_copy(x_hbm.at[cs(idx)], ins[b], sis[b]).wait()

            @pl.when(idx >= 2)
            def _():
                pltpu.make_async_copy(outs[b], out_hbm.at[cs(idx - 2)],
                                      sos[b]).wait()

            @plsc.parallel_loop(0, CHUNK, 1, unroll=1)
            def row_body(r):
                vs = [ins[b][r, pl.ds(i * L, L)] for i in range(ROW // L)]
                vs += [_INF, _INF]
                vs = _sort_row_vregs(vs)
                for i in range(ROW // L):
                    outs[b][r, pl.ds(i * L, L)] = vs[i]
            pltpu.async_copy(outs[b], out_hbm.at[cs(idx)], sos[b])
        return carry

    lax.fori_loop(0, NCHUNKS // 2, outer_body, 0)
    pltpu.make_async_copy(out0, out_hbm.at[cs(NCHUNKS - 2)], so0).wait()
    pltpu.make_async_copy(out1, out_hbm.at[cs(NCHUNKS - 1)], so1).wait()


@jax.jit
def kernel(x):
    x2 = x.reshape(R, ROW)
    mesh = plsc.VectorSubcoreMesh(core_axis_name="c", subcore_axis_name="s")
    out = pl.kernel(
        _sc_body,
        out_type=jax.ShapeDtypeStruct((R, ROW), jnp.float32),
        mesh=mesh,
        scratch_types=[
            pltpu.VMEM((CHUNK, ROW), jnp.float32),
            pltpu.VMEM((CHUNK, ROW), jnp.float32),
            pltpu.VMEM((CHUNK, ROW), jnp.float32),
            pltpu.VMEM((CHUNK, ROW), jnp.float32),
            pltpu.SemaphoreType.DMA,
            pltpu.SemaphoreType.DMA,
            pltpu.SemaphoreType.DMA,
            pltpu.SemaphoreType.DMA,
        ],
        compiler_params=pltpu.CompilerParams(needs_layout_passes=False),
    )(x2)
    return out.reshape(x.shape)
